# Initial kernel scaffold; baseline (speedup 1.0000x reference)
import jax
import jax.numpy as jnp
from jax.experimental import pallas as pl


def kernel(x, adj, W1, a1_src, a1_dst, W2, a2_src, a2_dst):
    raise NotImplementedError("write your pallas kernel here")


# speedup vs baseline: 1.8097x; 1.8097x over previous
import jax
import jax.numpy as jnp
from jax.experimental import pallas as pl
from jax.experimental.pallas import tpu as pltpu

N = 2048
INS = 512
CLASSES = 40
H1 = 8
O1 = 8
HD = H1 * O1
LEAK = 0.2
BR = 256
NB = N // BR
NEG = -1e9


def _layer1_kernel(x_ref, xT_ref, adj_ref, w_ref, wT_ref, asrc_ref, adstT_ref,
                   h1_ref, hall_s, fs_s, fdT_s):
    i = pl.program_id(0)

    @pl.when(i == 0)
    def _prologue():
        hall = jnp.dot(x_ref[...], w_ref[...],
                       preferred_element_type=jnp.float32)
        hall_s[...] = hall
        fs_s[...] = jnp.dot(hall, asrc_ref[...],
                            preferred_element_type=jnp.float32)
        hallT = jnp.dot(wT_ref[...], xT_ref[...],
                        preferred_element_type=jnp.float32)
        fdT_s[...] = jnp.dot(adstT_ref[...], hallT,
                             preferred_element_type=jnp.float32)

    adj = adj_ref[...]
    for h in range(H1):
        fs = fs_s[pl.ds(i * BR, BR), h:h + 1]
        fdT = fdT_s[h:h + 1, :]
        e = fs + fdT
        e = jnp.where(e >= 0, e, LEAK * e)
        e = jnp.where(adj > 0, e, jnp.float32(NEG))
        m = jnp.max(e, axis=1, keepdims=True)
        p = jnp.exp(e - m)
        s = jnp.sum(p, axis=1, keepdims=True)
        agg = jnp.dot(p, hall_s[:, h * O1:(h + 1) * O1],
                      preferred_element_type=jnp.float32) / s
        h1_ref[:, h * O1:(h + 1) * O1] = jnp.where(
            agg > 0, agg, jnp.exp(agg) - 1.0)


def _layer2_kernel(h1_ref, h1T_ref, adj_ref, w2_ref, vs_ref, vdT_ref,
                   out_ref, feat_s, fs_s, fdT_s):
    i = pl.program_id(0)

    @pl.when(i == 0)
    def _prologue():
        feat_s[...] = jnp.dot(h1_ref[...], w2_ref[...],
                              preferred_element_type=jnp.float32)
        fs_s[...] = jnp.dot(h1_ref[...], vs_ref[...],
                            preferred_element_type=jnp.float32)
        fdT_s[...] = jnp.dot(vdT_ref[...], h1T_ref[...],
                             preferred_element_type=jnp.float32)

    adj = adj_ref[...]
    fs = fs_s[pl.ds(i * BR, BR), :]
    e = fs + fdT_s[...]
    e = jnp.where(e >= 0, e, LEAK * e)
    e = jnp.where(adj > 0, e, jnp.float32(NEG))
    m = jnp.max(e, axis=1, keepdims=True)
    p = jnp.exp(e - m)
    s = jnp.sum(p, axis=1, keepdims=True)
    z = jnp.dot(p, feat_s[...],
                preferred_element_type=jnp.float32) / s
    m2 = jnp.max(z, axis=1, keepdims=True)
    lse = m2 + jnp.log(jnp.sum(jnp.exp(z - m2), axis=1, keepdims=True))
    out_ref[...] = z - lse


def kernel(x, adj, W1, a1_src, a1_dst, W2, a2_src, a2_dst):
    W1r = jnp.transpose(W1, (1, 0, 2)).reshape(INS, HD)
    W1rT = jnp.transpose(W1r)
    eye = jnp.eye(H1, dtype=jnp.float32)
    Asrc = (eye[:, None, :] * a1_src[:, :, None]).reshape(HD, H1)
    AdstT = (eye[:, :, None] * a1_dst[None, :, :]).reshape(H1, HD)
    xT = jnp.transpose(x)

    h1 = pl.pallas_call(
        _layer1_kernel,
        grid=(NB,),
        in_specs=[
            pl.BlockSpec((N, INS), lambda i: (0, 0)),
            pl.BlockSpec((INS, N), lambda i: (0, 0)),
            pl.BlockSpec((BR, N), lambda i: (i, 0)),
            pl.BlockSpec((INS, HD), lambda i: (0, 0)),
            pl.BlockSpec((HD, INS), lambda i: (0, 0)),
            pl.BlockSpec((HD, H1), lambda i: (0, 0)),
            pl.BlockSpec((H1, HD), lambda i: (0, 0)),
        ],
        out_specs=pl.BlockSpec((BR, HD), lambda i: (i, 0)),
        out_shape=jax.ShapeDtypeStruct((N, HD), jnp.float32),
        scratch_shapes=[
            pltpu.VMEM((N, HD), jnp.float32),
            pltpu.VMEM((N, H1), jnp.float32),
            pltpu.VMEM((H1, N), jnp.float32),
        ],
    )(x, xT, adj, W1r, W1rT, Asrc, AdstT)

    W2r = W2[0]
    vs = jnp.dot(W2r, a2_src[0])[:, None]
    vdT = jnp.dot(W2r, a2_dst[0])[None, :]
    h1T = jnp.transpose(h1)

    out = pl.pallas_call(
        _layer2_kernel,
        grid=(NB,),
        in_specs=[
            pl.BlockSpec((N, HD), lambda i: (0, 0)),
            pl.BlockSpec((HD, N), lambda i: (0, 0)),
            pl.BlockSpec((BR, N), lambda i: (i, 0)),
            pl.BlockSpec((HD, CLASSES), lambda i: (0, 0)),
            pl.BlockSpec((HD, 1), lambda i: (0, 0)),
            pl.BlockSpec((1, HD), lambda i: (0, 0)),
        ],
        out_specs=pl.BlockSpec((BR, CLASSES), lambda i: (i, 0)),
        out_shape=jax.ShapeDtypeStruct((N, CLASSES), jnp.float32),
        scratch_shapes=[
            pltpu.VMEM((N, CLASSES), jnp.float32),
            pltpu.VMEM((N, 1), jnp.float32),
            pltpu.VMEM((1, N), jnp.float32),
        ],
    )(h1, h1T, adj, W2r, vs, vdT)
    return out


# speedup vs baseline: 2.4203x; 1.3374x over previous
import jax
import jax.numpy as jnp
from jax.experimental import pallas as pl
from jax.experimental.pallas import tpu as pltpu

N = 2048
INS = 512
CLASSES = 40
H1 = 8
O1 = 8
HD = H1 * O1
LEAK = 0.2
BR = 256
NB = N // BR
HS = 16


def _layer1_kernel(x_ref, xT_ref, adj_ref, w_ref, wT_ref, asrc_ref, adstT_ref,
                   h1_ref, haug_s, fs_s, fdT_s, fm_s):
    i = pl.program_id(0)

    @pl.when(i == 0)
    def _prologue():
        hall = jnp.dot(x_ref[...], w_ref[...],
                       preferred_element_type=jnp.float32)
        fs_s[...] = jnp.dot(hall, asrc_ref[...],
                            preferred_element_type=jnp.float32)
        hallT = jnp.dot(wT_ref[...], xT_ref[...],
                        preferred_element_type=jnp.float32)
        fdT = jnp.dot(adstT_ref[...], hallT,
                      preferred_element_type=jnp.float32)
        fdT_s[...] = fdT
        fm_s[...] = jnp.max(fdT, axis=1, keepdims=True)
        haug_s[...] = jnp.zeros((N, H1 * HS), jnp.float32)
        for h in range(H1):
            haug_s[:, h * HS:h * HS + O1] = hall[:, h * O1:(h + 1) * O1]
            haug_s[:, h * HS + O1:h * HS + O1 + 1] = jnp.ones((N, 1),
                                                              jnp.float32)

    adj = adj_ref[...]
    for h in range(H1):
        fs = fs_s[pl.ds(i * BR, BR), h:h + 1]
        t = fs + fm_s[h:h + 1, :]
        m = jnp.maximum(t, LEAK * t)
        e = fs + fdT_s[h:h + 1, :]
        p = jnp.exp(jnp.maximum(e, LEAK * e) - m) * adj
        agg = jnp.dot(p, haug_s[:, h * HS:(h + 1) * HS],
                      preferred_element_type=jnp.float32)
        o = agg[:, 0:O1] / agg[:, O1:O1 + 1]
        h1_ref[:, h * O1:(h + 1) * O1] = jnp.where(
            o > 0, o, jnp.exp(o) - 1.0)


def _layer2_kernel(h1_ref, h1T_ref, adj_ref, w2_ref, vs_ref, vdT_ref,
                   out_ref, faug_s, fs_s, fdT_s, fm_s):
    i = pl.program_id(0)

    @pl.when(i == 0)
    def _prologue():
        faug_s[:, 0:CLASSES] = jnp.dot(h1_ref[...], w2_ref[...],
                                       preferred_element_type=jnp.float32)
        faug_s[:, CLASSES:CLASSES + 1] = jnp.ones((N, 1), jnp.float32)
        fs_s[...] = jnp.dot(h1_ref[...], vs_ref[...],
                            preferred_element_type=jnp.float32)
        fdT = jnp.dot(vdT_ref[...], h1T_ref[...],
                      preferred_element_type=jnp.float32)
        fdT_s[...] = fdT
        fm_s[...] = jnp.max(fdT, axis=1, keepdims=True)

    adj = adj_ref[...]
    fs = fs_s[pl.ds(i * BR, BR), :]
    t = fs + fm_s[...]
    m = jnp.maximum(t, LEAK * t)
    e = fs + fdT_s[...]
    p = jnp.exp(jnp.maximum(e, LEAK * e) - m) * adj
    agg = jnp.dot(p, faug_s[...],
                  preferred_element_type=jnp.float32)
    z = agg[:, 0:CLASSES] / agg[:, CLASSES:CLASSES + 1]
    m2 = jnp.max(z, axis=1, keepdims=True)
    lse = m2 + jnp.log(jnp.sum(jnp.exp(z - m2), axis=1, keepdims=True))
    out_ref[...] = z - lse


def kernel(x, adj, W1, a1_src, a1_dst, W2, a2_src, a2_dst):
    W1r = jnp.transpose(W1, (1, 0, 2)).reshape(INS, HD)
    W1rT = jnp.transpose(W1r)
    eye = jnp.eye(H1, dtype=jnp.float32)
    Asrc = (eye[:, None, :] * a1_src[:, :, None]).reshape(HD, H1)
    AdstT = (eye[:, :, None] * a1_dst[None, :, :]).reshape(H1, HD)
    xT = jnp.transpose(x)

    h1 = pl.pallas_call(
        _layer1_kernel,
        grid=(NB,),
        in_specs=[
            pl.BlockSpec((N, INS), lambda i: (0, 0)),
            pl.BlockSpec((INS, N), lambda i: (0, 0)),
            pl.BlockSpec((BR, N), lambda i: (i, 0)),
            pl.BlockSpec((INS, HD), lambda i: (0, 0)),
            pl.BlockSpec((HD, INS), lambda i: (0, 0)),
            pl.BlockSpec((HD, H1), lambda i: (0, 0)),
            pl.BlockSpec((H1, HD), lambda i: (0, 0)),
        ],
        out_specs=pl.BlockSpec((BR, HD), lambda i: (i, 0)),
        out_shape=jax.ShapeDtypeStruct((N, HD), jnp.float32),
        scratch_shapes=[
            pltpu.VMEM((N, H1 * HS), jnp.float32),
            pltpu.VMEM((N, H1), jnp.float32),
            pltpu.VMEM((H1, N), jnp.float32),
            pltpu.VMEM((H1, 1), jnp.float32),
        ],
    )(x, xT, adj, W1r, W1rT, Asrc, AdstT)

    W2r = W2[0]
    vs = jnp.dot(W2r, a2_src[0])[:, None]
    vdT = jnp.dot(W2r, a2_dst[0])[None, :]
    h1T = jnp.transpose(h1)

    out = pl.pallas_call(
        _layer2_kernel,
        grid=(NB,),
        in_specs=[
            pl.BlockSpec((N, HD), lambda i: (0, 0)),
            pl.BlockSpec((HD, N), lambda i: (0, 0)),
            pl.BlockSpec((BR, N), lambda i: (i, 0)),
            pl.BlockSpec((HD, CLASSES), lambda i: (0, 0)),
            pl.BlockSpec((HD, 1), lambda i: (0, 0)),
            pl.BlockSpec((1, HD), lambda i: (0, 0)),
        ],
        out_specs=pl.BlockSpec((BR, CLASSES), lambda i: (i, 0)),
        out_shape=jax.ShapeDtypeStruct((N, CLASSES), jnp.float32),
        scratch_shapes=[
            pltpu.VMEM((N, CLASSES + 1), jnp.float32),
            pltpu.VMEM((N, 1), jnp.float32),
            pltpu.VMEM((1, N), jnp.float32),
            pltpu.VMEM((1, 1), jnp.float32),
        ],
    )(h1, h1T, adj, W2r, vs, vdT)
    return out


# speedup vs baseline: 3.0080x; 1.2428x over previous
import jax
import jax.numpy as jnp
from jax.experimental import pallas as pl
from jax.experimental.pallas import tpu as pltpu

N = 2048
INS = 512
CLASSES = 40
H1 = 8
O1 = 8
HD = H1 * O1
LEAK = 0.2
BR = 256
NB = N // BR
HS = 16
LOG2E = 1.4426950408889634


def _layer1_kernel(x_ref, w_ref, asrc_ref, adst_ref, adj_ref,
                   h1_ref, haug_s, a_s, c_s, b_s, d_s):
    i = pl.program_id(0)

    @pl.when(i == 0)
    def _prologue():
        hall = jnp.dot(x_ref[...], w_ref[...],
                       preferred_element_type=jnp.float32)
        fs = jnp.dot(hall, asrc_ref[...],
                     preferred_element_type=jnp.float32)
        fd = jnp.dot(hall, adst_ref[...],
                     preferred_element_type=jnp.float32)
        fdT = jnp.transpose(fd)
        mh = (jnp.max(fs, axis=0, keepdims=True).T
              + jnp.max(fdT, axis=1, keepdims=True))
        mh = jnp.maximum(mh, LEAK * mh)
        a_s[...] = LOG2E * fs
        c_s[...] = (LEAK * LOG2E) * fs
        b_s[...] = LOG2E * fdT - LOG2E * mh
        d_s[...] = (LEAK * LOG2E) * fdT - LOG2E * mh
        haug_s[...] = jnp.zeros((N, H1 * HS), jnp.float32)
        for h in range(H1):
            haug_s[:, h * HS:h * HS + O1] = hall[:, h * O1:(h + 1) * O1]
            haug_s[:, h * HS + O1:h * HS + O1 + 1] = jnp.ones((N, 1),
                                                              jnp.float32)

    adj = adj_ref[...]
    for h in range(H1):
        a = a_s[pl.ds(i * BR, BR), h:h + 1] + b_s[h:h + 1, :]
        c = c_s[pl.ds(i * BR, BR), h:h + 1] + d_s[h:h + 1, :]
        p = jnp.exp2(jnp.maximum(a, c)) * adj
        agg = jnp.dot(p, haug_s[:, h * HS:(h + 1) * HS],
                      preferred_element_type=jnp.float32)
        o = agg[:, 0:O1] / agg[:, O1:O1 + 1]
        h1_ref[:, h * O1:(h + 1) * O1] = jnp.where(
            o > 0, o, jnp.exp(o) - 1.0)


def _layer2_kernel(h1_ref, adj_ref, w2_ref, vs_ref, vd_ref,
                   out_ref, faug_s, a_s, c_s, b_s, d_s):
    i = pl.program_id(0)

    @pl.when(i == 0)
    def _prologue():
        faug_s[:, 0:CLASSES] = jnp.dot(h1_ref[...], w2_ref[...],
                                       preferred_element_type=jnp.float32)
        faug_s[:, CLASSES:CLASSES + 1] = jnp.ones((N, 1), jnp.float32)
        fs = jnp.dot(h1_ref[...], vs_ref[...],
                     preferred_element_type=jnp.float32)
        fd = jnp.dot(h1_ref[...], vd_ref[...],
                     preferred_element_type=jnp.float32)
        fdT = jnp.transpose(fd)
        m = jnp.max(fs) + jnp.max(fdT)
        m = jnp.maximum(m, LEAK * m)
        a_s[...] = LOG2E * fs
        c_s[...] = (LEAK * LOG2E) * fs
        b_s[...] = LOG2E * fdT - LOG2E * m
        d_s[...] = (LEAK * LOG2E) * fdT - LOG2E * m

    adj = adj_ref[...]
    a = a_s[pl.ds(i * BR, BR), :] + b_s[...]
    c = c_s[pl.ds(i * BR, BR), :] + d_s[...]
    p = jnp.exp2(jnp.maximum(a, c)) * adj
    agg = jnp.dot(p, faug_s[...],
                  preferred_element_type=jnp.float32)
    z = agg[:, 0:CLASSES] / agg[:, CLASSES:CLASSES + 1]
    m2 = jnp.max(z, axis=1, keepdims=True)
    lse = m2 + jnp.log(jnp.sum(jnp.exp(z - m2), axis=1, keepdims=True))
    out_ref[...] = z - lse


def kernel(x, adj, W1, a1_src, a1_dst, W2, a2_src, a2_dst):
    W1r = jnp.transpose(W1, (1, 0, 2)).reshape(INS, HD)
    eye = jnp.eye(H1, dtype=jnp.float32)
    Asrc = (eye[:, None, :] * a1_src[:, :, None]).reshape(HD, H1)
    Adst = (eye[:, None, :] * a1_dst[:, :, None]).reshape(HD, H1)

    h1 = pl.pallas_call(
        _layer1_kernel,
        grid=(NB,),
        in_specs=[
            pl.BlockSpec((N, INS), lambda i: (0, 0)),
            pl.BlockSpec((INS, HD), lambda i: (0, 0)),
            pl.BlockSpec((HD, H1), lambda i: (0, 0)),
            pl.BlockSpec((HD, H1), lambda i: (0, 0)),
            pl.BlockSpec((BR, N), lambda i: (i, 0)),
        ],
        out_specs=pl.BlockSpec((BR, HD), lambda i: (i, 0)),
        out_shape=jax.ShapeDtypeStruct((N, HD), jnp.float32),
        scratch_shapes=[
            pltpu.VMEM((N, H1 * HS), jnp.float32),
            pltpu.VMEM((N, H1), jnp.float32),
            pltpu.VMEM((N, H1), jnp.float32),
            pltpu.VMEM((H1, N), jnp.float32),
            pltpu.VMEM((H1, N), jnp.float32),
        ],
    )(x, W1r, Asrc, Adst, adj)

    W2r = W2[0]
    vs = jnp.dot(W2r, a2_src[0])[:, None]
    vd = jnp.dot(W2r, a2_dst[0])[:, None]

    out = pl.pallas_call(
        _layer2_kernel,
        grid=(NB,),
        in_specs=[
            pl.BlockSpec((N, HD), lambda i: (0, 0)),
            pl.BlockSpec((BR, N), lambda i: (i, 0)),
            pl.BlockSpec((HD, CLASSES), lambda i: (0, 0)),
            pl.BlockSpec((HD, 1), lambda i: (0, 0)),
            pl.BlockSpec((HD, 1), lambda i: (0, 0)),
        ],
        out_specs=pl.BlockSpec((BR, CLASSES), lambda i: (i, 0)),
        out_shape=jax.ShapeDtypeStruct((N, CLASSES), jnp.float32),
        scratch_shapes=[
            pltpu.VMEM((N, CLASSES + 1), jnp.float32),
            pltpu.VMEM((N, 1), jnp.float32),
            pltpu.VMEM((N, 1), jnp.float32),
            pltpu.VMEM((1, N), jnp.float32),
            pltpu.VMEM((1, N), jnp.float32),
        ],
    )(h1, adj, W2r, vs, vd)
    return out


# speedup vs baseline: 3.4749x; 1.1552x over previous
import jax
import jax.numpy as jnp
from jax.experimental import pallas as pl
from jax.experimental.pallas import tpu as pltpu

N = 2048
INS = 512
CLASSES = 40
H1 = 8
O1 = 8
HD = H1 * O1
LEAK = 0.2
BR = 256
NB = N // BR
RG = BR // 8
FW = 72
FW2 = 48


def _layer1_kernel(x_ref, w_ref, asrc_ref, adst_ref, adj_ref, h1_ref,
                   haug_s, es_s, fs2_s, ed_s, fd2_s):
    i = pl.program_id(0)

    @pl.when(i == 0)
    def _prologue():
        hall = jnp.dot(x_ref[...], w_ref[...],
                       preferred_element_type=jnp.float32)
        fsrc = jnp.dot(hall, asrc_ref[...],
                       preferred_element_type=jnp.float32)
        fdst = jnp.dot(hall, adst_ref[...],
                       preferred_element_type=jnp.float32)
        alpha = jnp.max(fsrc, axis=0, keepdims=True)
        es_s[...] = jnp.exp(fsrc - alpha).reshape(N // 8, 8, H1)
        fs2_s[...] = jnp.exp(LEAK * fsrc - alpha).reshape(N // 8, 8, H1)
        fdT = jnp.transpose(fdst)
        beta = jnp.max(fdT, axis=1, keepdims=True)
        edT = jnp.exp(fdT - beta)
        fd2T = jnp.exp(LEAK * fdT - beta)
        for h in range(H1):
            ed_s[h, :, :] = jnp.broadcast_to(edT[h:h + 1, :], (8, N))
            fd2_s[h, :, :] = jnp.broadcast_to(fd2T[h:h + 1, :], (8, N))
        haug_s[:, 0:HD] = hall
        haug_s[:, HD:HD + 1] = jnp.ones((N, 1), jnp.float32)
        haug_s[:, HD + 1:FW] = jnp.zeros((N, FW - HD - 1), jnp.float32)

    adj = adj_ref[...].reshape(RG, 8, N)
    haug = haug_s[...]
    for h in range(H1):
        es = es_s[pl.ds(i * RG, RG), :, h:h + 1]
        fs2 = fs2_s[pl.ds(i * RG, RG), :, h:h + 1]
        u = es * ed_s[h:h + 1, :, :]
        v = fs2 * fd2_s[h:h + 1, :, :]
        p = (jnp.maximum(u, v) * adj).reshape(BR, N)
        agg = jnp.dot(p, haug,
                      preferred_element_type=jnp.float32)
        o = agg[:, h * O1:(h + 1) * O1] / agg[:, HD:HD + 1]
        h1_ref[:, h * O1:(h + 1) * O1] = jnp.where(
            o > 0, o, jnp.exp(o) - 1.0)


def _layer2_kernel(h1_ref, adj_ref, w2_ref, vs_ref, vd_ref, out_ref,
                   faug_s, es_s, fs2_s, ed_s, fd2_s):
    i = pl.program_id(0)

    @pl.when(i == 0)
    def _prologue():
        faug_s[:, 0:CLASSES] = jnp.dot(h1_ref[...], w2_ref[...],
                                       preferred_element_type=jnp.float32)
        faug_s[:, CLASSES:CLASSES + 1] = jnp.ones((N, 1), jnp.float32)
        faug_s[:, CLASSES + 1:FW2] = jnp.zeros((N, FW2 - CLASSES - 1),
                                               jnp.float32)
        fsrc = jnp.dot(h1_ref[...], vs_ref[...],
                       preferred_element_type=jnp.float32)
        fdst = jnp.dot(h1_ref[...], vd_ref[...],
                       preferred_element_type=jnp.float32)
        alpha = jnp.max(fsrc)
        es_s[...] = jnp.exp(fsrc - alpha).reshape(N // 8, 8, 1)
        fs2_s[...] = jnp.exp(LEAK * fsrc - alpha).reshape(N // 8, 8, 1)
        fdT = jnp.transpose(fdst)
        beta = jnp.max(fdT)
        ed_s[...] = jnp.broadcast_to(jnp.exp(fdT - beta), (8, N))[None]
        fd2_s[...] = jnp.broadcast_to(jnp.exp(LEAK * fdT - beta), (8, N))[None]

    adj = adj_ref[...].reshape(RG, 8, N)
    es = es_s[pl.ds(i * RG, RG), :, :]
    fs2 = fs2_s[pl.ds(i * RG, RG), :, :]
    u = es * ed_s[...]
    v = fs2 * fd2_s[...]
    p = (jnp.maximum(u, v) * adj).reshape(BR, N)
    agg = jnp.dot(p, faug_s[...],
                  preferred_element_type=jnp.float32)
    z = agg[:, 0:CLASSES] / agg[:, CLASSES:CLASSES + 1]
    m2 = jnp.max(z, axis=1, keepdims=True)
    lse = m2 + jnp.log(jnp.sum(jnp.exp(z - m2), axis=1, keepdims=True))
    out_ref[...] = z - lse


def kernel(x, adj, W1, a1_src, a1_dst, W2, a2_src, a2_dst):
    W1r = jnp.transpose(W1, (1, 0, 2)).reshape(INS, HD)
    eye = jnp.eye(H1, dtype=jnp.float32)
    Asrc = (eye[:, None, :] * a1_src[:, :, None]).reshape(HD, H1)
    Adst = (eye[:, None, :] * a1_dst[:, :, None]).reshape(HD, H1)

    h1 = pl.pallas_call(
        _layer1_kernel,
        grid=(NB,),
        in_specs=[
            pl.BlockSpec((N, INS), lambda i: (0, 0)),
            pl.BlockSpec((INS, HD), lambda i: (0, 0)),
            pl.BlockSpec((HD, H1), lambda i: (0, 0)),
            pl.BlockSpec((HD, H1), lambda i: (0, 0)),
            pl.BlockSpec((BR, N), lambda i: (i, 0)),
        ],
        out_specs=pl.BlockSpec((BR, HD), lambda i: (i, 0)),
        out_shape=jax.ShapeDtypeStruct((N, HD), jnp.float32),
        scratch_shapes=[
            pltpu.VMEM((N, FW), jnp.float32),
            pltpu.VMEM((N // 8, 8, H1), jnp.float32),
            pltpu.VMEM((N // 8, 8, H1), jnp.float32),
            pltpu.VMEM((H1, 8, N), jnp.float32),
            pltpu.VMEM((H1, 8, N), jnp.float32),
        ],
    )(x, W1r, Asrc, Adst, adj)

    W2r = W2[0]
    vs = jnp.dot(W2r, a2_src[0])[:, None]
    vd = jnp.dot(W2r, a2_dst[0])[:, None]

    out = pl.pallas_call(
        _layer2_kernel,
        grid=(NB,),
        in_specs=[
            pl.BlockSpec((N, HD), lambda i: (0, 0)),
            pl.BlockSpec((BR, N), lambda i: (i, 0)),
            pl.BlockSpec((HD, CLASSES), lambda i: (0, 0)),
            pl.BlockSpec((HD, 1), lambda i: (0, 0)),
            pl.BlockSpec((HD, 1), lambda i: (0, 0)),
        ],
        out_specs=pl.BlockSpec((BR, CLASSES), lambda i: (i, 0)),
        out_shape=jax.ShapeDtypeStruct((N, CLASSES), jnp.float32),
        scratch_shapes=[
            pltpu.VMEM((N, FW2), jnp.float32),
            pltpu.VMEM((N // 8, 8, 1), jnp.float32),
            pltpu.VMEM((N // 8, 8, 1), jnp.float32),
            pltpu.VMEM((1, 8, N), jnp.float32),
            pltpu.VMEM((1, 8, N), jnp.float32),
        ],
    )(h1, adj, W2r, vs, vd)
    return out


# speedup vs baseline: 3.5918x; 1.0336x over previous
import jax
import jax.numpy as jnp
from jax.experimental import pallas as pl
from jax.experimental.pallas import tpu as pltpu

N = 2048
INS = 512
CLASSES = 40
H1 = 8
O1 = 8
HD = H1 * O1
LEAK = 0.2
BR = 1024
NB = N // BR
RG = BR // 8
FW = 72
FW2 = 48


def _layer1_kernel(x_ref, w_ref, asrc_ref, adst_ref, adj_ref, h1_ref,
                   haug_s, es_s, fs2_s, ed_s, fd2_s):
    i = pl.program_id(0)

    @pl.when(i == 0)
    def _prologue():
        hall = jnp.dot(x_ref[...], w_ref[...],
                       preferred_element_type=jnp.float32)
        fsrc = jnp.dot(hall, asrc_ref[...],
                       preferred_element_type=jnp.float32)
        fdst = jnp.dot(hall, adst_ref[...],
                       preferred_element_type=jnp.float32)
        alpha = jnp.max(fsrc, axis=0, keepdims=True)
        es_s[...] = jnp.exp(fsrc - alpha).reshape(N // 8, 8, H1)
        fs2_s[...] = jnp.exp(LEAK * fsrc - alpha).reshape(N // 8, 8, H1)
        fdT = jnp.transpose(fdst)
        beta = jnp.max(fdT, axis=1, keepdims=True)
        edT = jnp.exp(fdT - beta)
        fd2T = jnp.exp(LEAK * fdT - beta)
        for h in range(H1):
            ed_s[h, :, :] = jnp.broadcast_to(edT[h:h + 1, :], (8, N))
            fd2_s[h, :, :] = jnp.broadcast_to(fd2T[h:h + 1, :], (8, N))
        haug_s[:, 0:HD] = hall
        haug_s[:, HD:HD + 1] = jnp.ones((N, 1), jnp.float32)
        haug_s[:, HD + 1:FW] = jnp.zeros((N, FW - HD - 1), jnp.float32)

    adj = adj_ref[...].reshape(RG, 8, N)
    haug = haug_s[...]
    for h in range(H1):
        es = es_s[pl.ds(i * RG, RG), :, h:h + 1]
        fs2 = fs2_s[pl.ds(i * RG, RG), :, h:h + 1]
        u = es * ed_s[h:h + 1, :, :]
        v = fs2 * fd2_s[h:h + 1, :, :]
        p = (jnp.maximum(u, v) * adj).reshape(BR, N)
        agg = jnp.dot(p, haug,
                      preferred_element_type=jnp.float32)
        o = agg[:, h * O1:(h + 1) * O1] / agg[:, HD:HD + 1]
        h1_ref[:, h * O1:(h + 1) * O1] = jnp.where(
            o > 0, o, jnp.exp(o) - 1.0)


def _layer2_kernel(h1_ref, adj_ref, w2_ref, vs_ref, vd_ref, out_ref,
                   faug_s, es_s, fs2_s, ed_s, fd2_s):
    i = pl.program_id(0)

    @pl.when(i == 0)
    def _prologue():
        faug_s[:, 0:CLASSES] = jnp.dot(h1_ref[...], w2_ref[...],
                                       preferred_element_type=jnp.float32)
        faug_s[:, CLASSES:CLASSES + 1] = jnp.ones((N, 1), jnp.float32)
        faug_s[:, CLASSES + 1:FW2] = jnp.zeros((N, FW2 - CLASSES - 1),
                                               jnp.float32)
        fsrc = jnp.dot(h1_ref[...], vs_ref[...],
                       preferred_element_type=jnp.float32)
        fdst = jnp.dot(h1_ref[...], vd_ref[...],
                       preferred_element_type=jnp.float32)
        alpha = jnp.max(fsrc)
        es_s[...] = jnp.exp(fsrc - alpha).reshape(N // 8, 8, 1)
        fs2_s[...] = jnp.exp(LEAK * fsrc - alpha).reshape(N // 8, 8, 1)
        fdT = jnp.transpose(fdst)
        beta = jnp.max(fdT)
        ed_s[...] = jnp.broadcast_to(jnp.exp(fdT - beta), (8, N))[None]
        fd2_s[...] = jnp.broadcast_to(jnp.exp(LEAK * fdT - beta), (8, N))[None]

    adj = adj_ref[...].reshape(RG, 8, N)
    es = es_s[pl.ds(i * RG, RG), :, :]
    fs2 = fs2_s[pl.ds(i * RG, RG), :, :]
    u = es * ed_s[...]
    v = fs2 * fd2_s[...]
    p = (jnp.maximum(u, v) * adj).reshape(BR, N)
    agg = jnp.dot(p, faug_s[...],
                  preferred_element_type=jnp.float32)
    z = agg[:, 0:CLASSES] / agg[:, CLASSES:CLASSES + 1]
    m2 = jnp.max(z, axis=1, keepdims=True)
    lse = m2 + jnp.log(jnp.sum(jnp.exp(z - m2), axis=1, keepdims=True))
    out_ref[...] = z - lse


def kernel(x, adj, W1, a1_src, a1_dst, W2, a2_src, a2_dst):
    W1r = jnp.transpose(W1, (1, 0, 2)).reshape(INS, HD)
    eye = jnp.eye(H1, dtype=jnp.float32)
    Asrc = (eye[:, None, :] * a1_src[:, :, None]).reshape(HD, H1)
    Adst = (eye[:, None, :] * a1_dst[:, :, None]).reshape(HD, H1)

    h1 = pl.pallas_call(
        _layer1_kernel,
        grid=(NB,),
        in_specs=[
            pl.BlockSpec((N, INS), lambda i: (0, 0)),
            pl.BlockSpec((INS, HD), lambda i: (0, 0)),
            pl.BlockSpec((HD, H1), lambda i: (0, 0)),
            pl.BlockSpec((HD, H1), lambda i: (0, 0)),
            pl.BlockSpec((BR, N), lambda i: (i, 0)),
        ],
        out_specs=pl.BlockSpec((BR, HD), lambda i: (i, 0)),
        out_shape=jax.ShapeDtypeStruct((N, HD), jnp.float32),
        scratch_shapes=[
            pltpu.VMEM((N, FW), jnp.float32),
            pltpu.VMEM((N // 8, 8, H1), jnp.float32),
            pltpu.VMEM((N // 8, 8, H1), jnp.float32),
            pltpu.VMEM((H1, 8, N), jnp.float32),
            pltpu.VMEM((H1, 8, N), jnp.float32),
        ],
    )(x, W1r, Asrc, Adst, adj)

    W2r = W2[0]
    vs = jnp.dot(W2r, a2_src[0])[:, None]
    vd = jnp.dot(W2r, a2_dst[0])[:, None]

    out = pl.pallas_call(
        _layer2_kernel,
        grid=(NB,),
        in_specs=[
            pl.BlockSpec((N, HD), lambda i: (0, 0)),
            pl.BlockSpec((BR, N), lambda i: (i, 0)),
            pl.BlockSpec((HD, CLASSES), lambda i: (0, 0)),
            pl.BlockSpec((HD, 1), lambda i: (0, 0)),
            pl.BlockSpec((HD, 1), lambda i: (0, 0)),
        ],
        out_specs=pl.BlockSpec((BR, CLASSES), lambda i: (i, 0)),
        out_shape=jax.ShapeDtypeStruct((N, CLASSES), jnp.float32),
        scratch_shapes=[
            pltpu.VMEM((N, FW2), jnp.float32),
            pltpu.VMEM((N // 8, 8, 1), jnp.float32),
            pltpu.VMEM((N // 8, 8, 1), jnp.float32),
            pltpu.VMEM((1, 8, N), jnp.float32),
            pltpu.VMEM((1, 8, N), jnp.float32),
        ],
    )(h1, adj, W2r, vs, vd)
    return out


# speedup vs baseline: 3.9675x; 1.1046x over previous
import jax
import jax.numpy as jnp
from jax.experimental import pallas as pl
from jax.experimental.pallas import tpu as pltpu

N = 2048
INS = 512
CLASSES = 40
H1 = 8
O1 = 8
HD = H1 * O1
LEAK = 0.2
BR = 512
NB2 = N // BR
RG = BR // 8
FW = 72
FW2 = 48


def _fused_kernel(x_ref, w_ref, asrc_ref, adst_ref, adj_ref,
                  w2_ref, vs_ref, vd_ref, out_ref,
                  adj_s, h1_s, haug_s, es_s, fs2_s, ed_s, fd2_s,
                  faug_s, es2_s, fs22_s, ed2_s, fd22_s):
    t = pl.program_id(0)

    @pl.when(t == 0)
    def _l1_prologue():
        hall = jnp.dot(x_ref[...], w_ref[...],
                       preferred_element_type=jnp.float32)
        fsrc = jnp.dot(hall, asrc_ref[...],
                       preferred_element_type=jnp.float32)
        fdst = jnp.dot(hall, adst_ref[...],
                       preferred_element_type=jnp.float32)
        alpha = jnp.max(fsrc, axis=0, keepdims=True)
        es_s[...] = jnp.exp(fsrc - alpha).reshape(N // 8, 8, H1)
        fs2_s[...] = jnp.exp(LEAK * fsrc - alpha).reshape(N // 8, 8, H1)
        fdT = jnp.transpose(fdst)
        beta = jnp.max(fdT, axis=1, keepdims=True)
        edT = jnp.exp(fdT - beta)
        fd2T = jnp.exp(LEAK * fdT - beta)
        for h in range(H1):
            ed_s[h, :, :] = jnp.broadcast_to(edT[h:h + 1, :], (8, N))
            fd2_s[h, :, :] = jnp.broadcast_to(fd2T[h:h + 1, :], (8, N))
        haug_s[:, 0:HD] = hall
        haug_s[:, HD:HD + 1] = jnp.ones((N, 1), jnp.float32)
        haug_s[:, HD + 1:FW] = jnp.zeros((N, FW - HD - 1), jnp.float32)

    @pl.when(t < NB2)
    def _l1_body():
        adjb = adj_ref[...]
        adj_s[pl.ds(t * BR, BR), :] = adjb
        adj3 = adjb.reshape(RG, 8, N)
        haug = haug_s[...]
        for h in range(H1):
            es = es_s[pl.ds(t * RG, RG), :, h:h + 1]
            fs2 = fs2_s[pl.ds(t * RG, RG), :, h:h + 1]
            u = es * ed_s[h:h + 1, :, :]
            v = fs2 * fd2_s[h:h + 1, :, :]
            p = (jnp.maximum(u, v) * adj3).reshape(BR, N)
            agg = jnp.dot(p, haug,
                          preferred_element_type=jnp.float32)
            o = agg[:, h * O1:(h + 1) * O1] / agg[:, HD:HD + 1]
            h1_s[pl.ds(t * BR, BR), h * O1:(h + 1) * O1] = jnp.where(
                o > 0, o, jnp.exp(o) - 1.0)

    @pl.when(t == NB2)
    def _l2_prologue():
        h1 = h1_s[...]
        faug_s[:, 0:CLASSES] = jnp.dot(h1, w2_ref[...],
                                       preferred_element_type=jnp.float32)
        faug_s[:, CLASSES:CLASSES + 1] = jnp.ones((N, 1), jnp.float32)
        faug_s[:, CLASSES + 1:FW2] = jnp.zeros((N, FW2 - CLASSES - 1),
                                               jnp.float32)
        fsrc = jnp.dot(h1, vs_ref[...],
                       preferred_element_type=jnp.float32)
        fdst = jnp.dot(h1, vd_ref[...],
                       preferred_element_type=jnp.float32)
        alpha = jnp.max(fsrc)
        es2_s[...] = jnp.exp(fsrc - alpha).reshape(N // 8, 8, 1)
        fs22_s[...] = jnp.exp(LEAK * fsrc - alpha).reshape(N // 8, 8, 1)
        fdT = jnp.transpose(fdst)
        beta = jnp.max(fdT)
        ed2_s[...] = jnp.broadcast_to(jnp.exp(fdT - beta), (8, N))[None]
        fd22_s[...] = jnp.broadcast_to(jnp.exp(LEAK * fdT - beta),
                                       (8, N))[None]

    @pl.when(t >= NB2)
    def _l2_body():
        j = t - NB2
        adj3 = adj_s[pl.ds(j * BR, BR), :].reshape(RG, 8, N)
        es = es2_s[pl.ds(j * RG, RG), :, :]
        fs2 = fs22_s[pl.ds(j * RG, RG), :, :]
        u = es * ed2_s[...]
        v = fs2 * fd22_s[...]
        p = (jnp.maximum(u, v) * adj3).reshape(BR, N)
        agg = jnp.dot(p, faug_s[...],
                      preferred_element_type=jnp.float32)
        z = agg[:, 0:CLASSES] / agg[:, CLASSES:CLASSES + 1]
        m2 = jnp.max(z, axis=1, keepdims=True)
        lse = m2 + jnp.log(jnp.sum(jnp.exp(z - m2), axis=1, keepdims=True))
        out_ref[...] = z - lse


def kernel(x, adj, W1, a1_src, a1_dst, W2, a2_src, a2_dst):
    W1r = jnp.transpose(W1, (1, 0, 2)).reshape(INS, HD)
    eye = jnp.eye(H1, dtype=jnp.float32)
    Asrc = (eye[:, None, :] * a1_src[:, :, None]).reshape(HD, H1)
    Adst = (eye[:, None, :] * a1_dst[:, :, None]).reshape(HD, H1)
    W2r = W2[0]
    vs = jnp.dot(W2r, a2_src[0])[:, None]
    vd = jnp.dot(W2r, a2_dst[0])[:, None]

    out = pl.pallas_call(
        _fused_kernel,
        grid=(2 * NB2,),
        in_specs=[
            pl.BlockSpec((N, INS), lambda t: (0, 0)),
            pl.BlockSpec((INS, HD), lambda t: (0, 0)),
            pl.BlockSpec((HD, H1), lambda t: (0, 0)),
            pl.BlockSpec((HD, H1), lambda t: (0, 0)),
            pl.BlockSpec((BR, N), lambda t: (jnp.minimum(t, NB2 - 1), 0)),
            pl.BlockSpec((HD, CLASSES), lambda t: (0, 0)),
            pl.BlockSpec((HD, 1), lambda t: (0, 0)),
            pl.BlockSpec((HD, 1), lambda t: (0, 0)),
        ],
        out_specs=pl.BlockSpec(
            (BR, CLASSES), lambda t: (jnp.where(t < NB2, t, t - NB2), 0)),
        out_shape=jax.ShapeDtypeStruct((N, CLASSES), jnp.float32),
        scratch_shapes=[
            pltpu.VMEM((N, N), jnp.float32),
            pltpu.VMEM((N, HD), jnp.float32),
            pltpu.VMEM((N, FW), jnp.float32),
            pltpu.VMEM((N // 8, 8, H1), jnp.float32),
            pltpu.VMEM((N // 8, 8, H1), jnp.float32),
            pltpu.VMEM((H1, 8, N), jnp.float32),
            pltpu.VMEM((H1, 8, N), jnp.float32),
            pltpu.VMEM((N, FW2), jnp.float32),
            pltpu.VMEM((N // 8, 8, 1), jnp.float32),
            pltpu.VMEM((N // 8, 8, 1), jnp.float32),
            pltpu.VMEM((1, 8, N), jnp.float32),
            pltpu.VMEM((1, 8, N), jnp.float32),
        ],
    )(x, W1r, Asrc, Adst, adj, W2r, vs, vd)
    return out
